# baseline (speedup 1.0000x reference)
import jax
import jax.numpy as jnp
from jax.experimental import pallas as pl
from jax.experimental.pallas import tpu as pltpu

BR = 512


def _rownorm_kernel(z_ref, out_ref):
    z = z_ref[...]
    s = z * z
    ones = jnp.ones((z.shape[1], 1), dtype=jnp.float32)
    out_ref[...] = -0.5 * jax.lax.dot_general(
        s, ones, (((1,), (0,)), ((), ())), preferred_element_type=jnp.float32
    )


def kernel(Z, labels):
    del labels
    n, d = Z.shape
    out = pl.pallas_call(
        _rownorm_kernel,
        grid=(n // BR,),
        in_specs=[pl.BlockSpec((BR, d), lambda i: (i, 0))],
        out_specs=pl.BlockSpec((BR, 1), lambda i: (i, 0)),
        out_shape=jax.ShapeDtypeStruct((n, 1), Z.dtype),
        compiler_params=pltpu.CompilerParams(
            dimension_semantics=("arbitrary",),
        ),
    )(Z)
    return out.reshape(n)


# speedup vs baseline: 1.0011x; 1.0011x over previous
import jax
import jax.numpy as jnp
from jax.experimental import pallas as pl
from jax.experimental.pallas import tpu as pltpu

BR = 512


def _rownorm_kernel(z_ref, out_ref):
    z = z_ref[...]
    s = z * z
    out_ref[...] = -0.5 * jnp.sum(s, axis=1, keepdims=True)


def kernel(Z, labels):
    del labels
    n, d = Z.shape
    out = pl.pallas_call(
        _rownorm_kernel,
        grid=(n // BR,),
        in_specs=[pl.BlockSpec((BR, d), lambda i: (i, 0))],
        out_specs=pl.BlockSpec((BR, 1), lambda i: (i, 0)),
        out_shape=jax.ShapeDtypeStruct((n, 1), Z.dtype),
        compiler_params=pltpu.CompilerParams(
            dimension_semantics=("arbitrary",),
        ),
    )(Z)
    return out.reshape(n)


# speedup vs baseline: 1.9201x; 1.9180x over previous
import jax
import jax.numpy as jnp
from jax.experimental import pallas as pl
from jax.experimental.pallas import tpu as pltpu

N, D = 16384, 128
CH = 1024
NBUF = 4
NCH = N // CH


def _rownorm_kernel(z_hbm, out_hbm, b0, b1, b2, b3, ovm, s0, s1, s2, s3, osem):
    bufs = (b0, b1, b2, b3)
    sems = (s0, s1, s2, s3)
    for b in range(NBUF):
        pltpu.make_async_copy(z_hbm.at[pl.ds(b * CH, CH)], bufs[b], sems[b]).start()
    for c in range(NCH):
        i = c % NBUF
        pltpu.make_async_copy(z_hbm.at[pl.ds(c * CH, CH)], bufs[i], sems[i]).wait()
        z = bufs[i][...]
        ovm[pl.ds(c * CH, CH), :] = -0.5 * jnp.sum(z * z, axis=1, keepdims=True)
        nxt = c + NBUF
        if nxt < NCH:
            pltpu.make_async_copy(
                z_hbm.at[pl.ds(nxt * CH, CH)], bufs[i], sems[i]
            ).start()
    out_copy = pltpu.make_async_copy(ovm, out_hbm, osem)
    out_copy.start()
    out_copy.wait()


def kernel(Z, labels):
    del labels
    out = pl.pallas_call(
        _rownorm_kernel,
        in_specs=[pl.BlockSpec(memory_space=pltpu.HBM)],
        out_specs=pl.BlockSpec(memory_space=pltpu.HBM),
        out_shape=jax.ShapeDtypeStruct((N, 1), Z.dtype),
        scratch_shapes=[
            pltpu.VMEM((CH, D), jnp.float32),
            pltpu.VMEM((CH, D), jnp.float32),
            pltpu.VMEM((CH, D), jnp.float32),
            pltpu.VMEM((CH, D), jnp.float32),
            pltpu.VMEM((N, 1), jnp.float32),
            pltpu.SemaphoreType.DMA,
            pltpu.SemaphoreType.DMA,
            pltpu.SemaphoreType.DMA,
            pltpu.SemaphoreType.DMA,
            pltpu.SemaphoreType.DMA,
        ],
    )(Z)
    return out.reshape(N)


# speedup vs baseline: 2.1981x; 1.1448x over previous
import jax
import jax.numpy as jnp
from jax.experimental import pallas as pl
from jax.experimental.pallas import tpu as pltpu

N, D = 16384, 128
NS_ = 4
BR = 512
NBLK = N // (NS_ * BR)


def _rownorm_kernel(*refs):
    z_refs, out_refs = refs[:NS_], refs[NS_:]
    for z_ref, out_ref in zip(z_refs, out_refs):
        z = z_ref[...]
        out_ref[...] = -0.5 * jnp.sum(z * z, axis=1)


def kernel(Z, labels):
    del labels

    def in_spec(k):
        return pl.BlockSpec((BR, D), lambda i, k=k: (k * NBLK + i, 0))

    outs = pl.pallas_call(
        _rownorm_kernel,
        grid=(NBLK,),
        in_specs=[in_spec(k) for k in range(NS_)],
        out_specs=[pl.BlockSpec((BR,), lambda i: (i,)) for _ in range(NS_)],
        out_shape=[jax.ShapeDtypeStruct((N // NS_,), Z.dtype) for _ in range(NS_)],
        compiler_params=pltpu.CompilerParams(
            dimension_semantics=("arbitrary",),
        ),
    )(*([Z] * NS_))
    return jnp.concatenate(outs, axis=0)


# speedup vs baseline: 2.5343x; 1.1530x over previous
import jax
import jax.numpy as jnp
from jax.experimental import pallas as pl
from jax.experimental.pallas import tpu as pltpu

N, D = 16384, 128
CH = 1024
NCH = N // CH


def _rownorm_kernel(z_hbm, out_hbm, zvm, ovm, osem, *sems):
    copies = [
        pltpu.make_async_copy(
            z_hbm.at[pl.ds(c * CH, CH)], zvm.at[pl.ds(c * CH, CH)], sems[c]
        )
        for c in range(NCH)
    ]
    for cp in copies:
        cp.start()
    for c in range(NCH):
        copies[c].wait()
        z = zvm[pl.ds(c * CH, CH), :]
        ovm[pl.ds(c * CH, CH)] = -0.5 * jnp.sum(z * z, axis=1)
    out_copy = pltpu.make_async_copy(ovm, out_hbm, osem)
    out_copy.start()
    out_copy.wait()


def kernel(Z, labels):
    del labels
    return pl.pallas_call(
        _rownorm_kernel,
        in_specs=[pl.BlockSpec(memory_space=pltpu.HBM)],
        out_specs=pl.BlockSpec(memory_space=pltpu.HBM),
        out_shape=jax.ShapeDtypeStruct((N,), Z.dtype),
        scratch_shapes=(
            [
                pltpu.VMEM((N, D), jnp.float32),
                pltpu.VMEM((N,), jnp.float32),
            ]
            + [pltpu.SemaphoreType.DMA] * (NCH + 1)
        ),
    )(Z)


# speedup vs baseline: 2.5524x; 1.0071x over previous
import jax
import jax.numpy as jnp
from jax.experimental import pallas as pl
from jax.experimental.pallas import tpu as pltpu

N, D = 16384, 128
CH = 1024
NBUF = 4
NCH = N // CH


def _rownorm_kernel(z_hbm, out_hbm, b0, b1, b2, b3, ovm, s0, s1, s2, s3, osem):
    bufs = (b0, b1, b2, b3)
    sems = (s0, s1, s2, s3)
    for b in range(NBUF):
        pltpu.make_async_copy(z_hbm.at[pl.ds(b * CH, CH)], bufs[b], sems[b]).start()
    for c in range(NCH):
        i = c % NBUF
        pltpu.make_async_copy(z_hbm.at[pl.ds(c * CH, CH)], bufs[i], sems[i]).wait()
        z = bufs[i][...]
        ovm[pl.ds(c * CH, CH)] = -0.5 * jnp.sum(z * z, axis=1)
        nxt = c + NBUF
        if nxt < NCH:
            pltpu.make_async_copy(
                z_hbm.at[pl.ds(nxt * CH, CH)], bufs[i], sems[i]
            ).start()
    out_copy = pltpu.make_async_copy(ovm, out_hbm, osem)
    out_copy.start()
    out_copy.wait()


def kernel(Z, labels):
    del labels
    return pl.pallas_call(
        _rownorm_kernel,
        in_specs=[pl.BlockSpec(memory_space=pltpu.HBM)],
        out_specs=pl.BlockSpec(memory_space=pltpu.HBM),
        out_shape=jax.ShapeDtypeStruct((N,), Z.dtype),
        scratch_shapes=[
            pltpu.VMEM((CH, D), jnp.float32),
            pltpu.VMEM((CH, D), jnp.float32),
            pltpu.VMEM((CH, D), jnp.float32),
            pltpu.VMEM((CH, D), jnp.float32),
            pltpu.VMEM((N,), jnp.float32),
            pltpu.SemaphoreType.DMA,
            pltpu.SemaphoreType.DMA,
            pltpu.SemaphoreType.DMA,
            pltpu.SemaphoreType.DMA,
            pltpu.SemaphoreType.DMA,
        ],
    )(Z)
